# baseline (speedup 1.0000x reference)
import functools

import jax
import jax.numpy as jnp
from jax import lax
from jax.experimental import pallas as pl
from jax.experimental.pallas import tpu as pltpu
from jax.experimental.pallas import tpu_sc as plsc

NC = 2
NS = 16
LANES = 16
NW = NC * NS


def _sc_kernel(B, L, C, V, interpret=False):
    rows_w = B // NW
    groups = rows_w // LANES
    pairs = (C + 1) // 2

    mesh = plsc.VectorSubcoreMesh(
        core_axis_name="c", subcore_axis_name="s", num_cores=NC, num_subcores=NS
    )

    @functools.partial(
        pl.kernel,
        out_type=jax.ShapeDtypeStruct((B, C), jnp.float32),
        mesh=mesh,
        scratch_types=[
            pltpu.VMEM((rows_w, L), jnp.int32),
            pltpu.VMEM((pairs * V,), jnp.int32),
            pltpu.VMEM((LANES,), jnp.float32),
            pltpu.VMEM((rows_w, C), jnp.float32),
        ],
        compiler_params=pltpu.CompilerParams(needs_layout_passes=False, use_tc_tiling_on_sc=False),
        interpret=interpret,
    )
    def run(ids_hbm, w_hbm, b_hbm, out_hbm, ids_v, w_v, b_v, out_v):
        wid = lax.axis_index("s") * NC + lax.axis_index("c")
        base = wid * rows_w

        pltpu.sync_copy(ids_hbm.at[pl.ds(base, rows_w)], ids_v)
        pltpu.sync_copy(w_hbm, w_v)
        pltpu.sync_copy(b_hbm, b_v)

        lane = lax.iota(jnp.int32, LANES)
        bvec = b_v[...]

        def group_body(g, _):
            rv = g * LANES + lane

            def tok_body(l, accs):
                lv = jnp.full((LANES,), l, jnp.int32)
                idv = plsc.load_gather(ids_v, [rv, lv])
                new = list(accs)
                for p in range(pairs):
                    x = plsc.load_gather(w_v, [idv + (p * V)])
                    new[2 * p] = new[2 * p] + plsc.bitcast(x << 16, jnp.float32)
                    if 2 * p + 1 < C:
                        new[2 * p + 1] = new[2 * p + 1] + plsc.bitcast(
                            x & jnp.int32(-65536), jnp.float32
                        )
                return tuple(new)

            init = tuple(jnp.full((LANES,), bvec[c], jnp.float32) for c in range(C))
            accs = lax.fori_loop(0, L, tok_body, init, unroll=8)

            m = accs[0]
            for c in range(1, C):
                m = jnp.maximum(m, accs[c])
            es = tuple(jnp.exp(acc - m) for acc in accs)
            s = es[0]
            for c in range(1, C):
                s = s + es[c]
            inv = jnp.float32(1.0) / s
            for c in range(C):
                plsc.store_scatter(
                    out_v, [rv, jnp.full((LANES,), c, jnp.int32)], es[c] * inv
                )
            return ()

        lax.fori_loop(0, groups, group_body, ())
        pltpu.sync_copy(out_v, out_hbm.at[pl.ds(base, rows_w)])

    return run


def kernel(ids, W, b):
    B, L = ids.shape
    C, V = W.shape
    b_pad = jnp.zeros((LANES,), jnp.float32).at[:C].set(b)
    pairs = (C + 1) // 2
    wb = jnp.zeros((2 * pairs, V), jnp.bfloat16).at[:C].set(W.astype(jnp.bfloat16))
    u = lax.bitcast_convert_type(wb, jnp.uint16).astype(jnp.uint32)
    packed = (u[0::2] | (u[1::2] << 16)).astype(jnp.int32)
    return _sc_kernel(B, L, C, V)(ids, packed.reshape(-1), b_pad)


# speedup vs baseline: 1.3443x; 1.3443x over previous
import functools

import jax
import jax.numpy as jnp
from jax import lax
from jax.experimental import pallas as pl
from jax.experimental.pallas import tpu as pltpu
from jax.experimental.pallas import tpu_sc as plsc

NC = 2
NS = 16
LANES = 16
NW = NC * NS


def _pick_window(n):
    for w in range(8, 0, -1):
        if n % w == 0:
            return w
    return 1


def _sc_kernel(B, L, C, V, interpret=False):
    rows_w = B // NW
    groups = rows_w // LANES
    pairs = (C + 1) // 2
    window = _pick_window(L)
    n_windows = L // window

    mesh = plsc.VectorSubcoreMesh(
        core_axis_name="c", subcore_axis_name="s", num_cores=NC, num_subcores=NS
    )

    @functools.partial(
        pl.kernel,
        out_type=jax.ShapeDtypeStruct((B, C), jnp.float32),
        mesh=mesh,
        scratch_types=[
            pltpu.VMEM((rows_w * L,), jnp.int32),
            pltpu.VMEM((pairs * V + LANES,), jnp.int32),
            pltpu.VMEM((rows_w, C), jnp.float32),
            pltpu.SemaphoreType.DMA,
            pltpu.SemaphoreType.DMA,
        ],
        compiler_params=pltpu.CompilerParams(
            needs_layout_passes=False, use_tc_tiling_on_sc=False
        ),
        interpret=interpret,
    )
    def run(ids_hbm, w_hbm, out_hbm, ids_v, w_v, out_v, sem1, sem2):
        wid = lax.axis_index("s") * NC + lax.axis_index("c")
        base = wid * rows_w
        half = rows_w * L // 2

        cp1 = pltpu.async_copy(
            ids_hbm.at[pl.ds(base * L, half)], ids_v.at[pl.ds(0, half)], sem1
        )
        cp2 = pltpu.async_copy(
            ids_hbm.at[pl.ds(base * L + half, half)],
            ids_v.at[pl.ds(half, half)],
            sem2,
        )
        pltpu.sync_copy(w_hbm, w_v)

        lane = lax.iota(jnp.int32, LANES)
        bvec = plsc.bitcast(w_v[pl.ds(pairs * V, LANES)], jnp.float32)
        zerob = jnp.zeros((2 * LANES,), jnp.bfloat16)

        def group_body(g, _):
            rv = g * LANES + lane
            rid = rv * L

            def win_body(w, accs):
                l0 = w * window
                accb = [zerob] * pairs
                for j in range(window):
                    idv = plsc.load_gather(ids_v, [rid + (l0 + j)])
                    for p in range(pairs):
                        x = plsc.load_gather(w_v, [idv + (p * V)])
                        accb[p] = accb[p] + plsc.bitcast(x, jnp.bfloat16)
                new = list(accs)
                for p in range(pairs):
                    xi = plsc.bitcast(accb[p], jnp.int32)
                    new[2 * p] = new[2 * p] + plsc.bitcast(xi << 16, jnp.float32)
                    if 2 * p + 1 < C:
                        new[2 * p + 1] = new[2 * p + 1] + plsc.bitcast(
                            xi & jnp.int32(-65536), jnp.float32
                        )
                return tuple(new)

            init = tuple(jnp.full((LANES,), bvec[c], jnp.float32) for c in range(C))
            accs = lax.fori_loop(0, n_windows, win_body, init)

            m = accs[0]
            for c in range(1, C):
                m = jnp.maximum(m, accs[c])
            es = tuple(jnp.exp(acc - m) for acc in accs)
            s = es[0]
            for c in range(1, C):
                s = s + es[c]
            inv = jnp.float32(1.0) / s
            for c in range(C):
                plsc.store_scatter(
                    out_v, [rv, jnp.full((LANES,), c, jnp.int32)], es[c] * inv
                )
            return ()

        cp1.wait()
        lax.fori_loop(0, groups // 2, group_body, ())
        cp2.wait()
        lax.fori_loop(groups // 2, groups, group_body, ())
        pltpu.sync_copy(out_v, out_hbm.at[pl.ds(base, rows_w)])

    return run


def kernel(ids, W, b):
    B, L = ids.shape
    C, V = W.shape
    pairs = (C + 1) // 2
    wb = jnp.zeros((2 * pairs, V), jnp.bfloat16).at[:C].set(W.astype(jnp.bfloat16))
    u = lax.bitcast_convert_type(wb, jnp.uint16).astype(jnp.uint32)
    packed = (u[0::2] | (u[1::2] << 16)).astype(jnp.int32)
    b_pad = jnp.zeros((LANES,), jnp.float32).at[:C].set(b)
    table = jnp.concatenate(
        [packed.reshape(-1), lax.bitcast_convert_type(b_pad, jnp.int32)]
    )
    out = _sc_kernel(B, L, C, V)(ids.reshape(-1), table)
    return out
